# Initial kernel scaffold; baseline (speedup 1.0000x reference)
import jax
import jax.numpy as jnp
from jax.experimental import pallas as pl


def kernel(hidden_states, Wg, gate_A, gate_B, up_A, up_B, down_A, down_B):
    raise NotImplementedError("write your pallas kernel here")


# speedup vs baseline: 1.5743x; 1.5743x over previous
import jax
import jax.numpy as jnp
from jax.experimental import pallas as pl
from jax.experimental.pallas import tpu as pltpu

B, S, H = 1, 2048, 1024
I, R, E, K = 1024, 256, 8, 2
TM = 512
NT = S // TM


def _moe_body(x_ref, wg_ref, ga_ref, gb_ref, ua_ref, ub_ref, da_ref, db_ref,
              out_ref, logits_ref):
    e = pl.program_id(1)
    x = x_ref[...]

    logits = jax.lax.dot_general(
        x, wg_ref[...], (((1,), (1,)), ((), ())),
        preferred_element_type=jnp.float32)
    p = jax.nn.softmax(logits, axis=-1)
    lane = jax.lax.broadcasted_iota(jnp.int32, (TM, E), 1)
    v1 = jnp.max(p, axis=-1, keepdims=True)
    idx1 = jnp.min(jnp.where(p == v1, lane, E), axis=-1, keepdims=True)
    p2 = jnp.where(lane == idx1, -1.0, p)
    v2 = jnp.max(p2, axis=-1, keepdims=True)
    idx2 = jnp.min(jnp.where(p2 == v2, lane, E), axis=-1, keepdims=True)
    denom = v1 + v2
    wcol = (jnp.where(idx1 == e, v1, 0.0) + jnp.where(idx2 == e, v2, 0.0)) / denom

    @pl.when(e == 0)
    def _():
        logits_ref[...] = logits
        out_ref[...] = jnp.zeros_like(out_ref)

    ga = ga_ref[0]
    gb = gb_ref[0]
    ua = ua_ref[0]
    ub = ub_ref[0]
    da = da_ref[0]
    db = db_ref[0]

    a_g = jax.lax.dot_general(x, ga, (((1,), (1,)), ((), ())),
                              preferred_element_type=jnp.float32)
    g = jax.lax.dot_general(a_g, gb, (((1,), (1,)), ((), ())),
                            preferred_element_type=jnp.float32)
    a_u = jax.lax.dot_general(x, ua, (((1,), (1,)), ((), ())),
                              preferred_element_type=jnp.float32)
    u = jax.lax.dot_general(a_u, ub, (((1,), (1,)), ((), ())),
                            preferred_element_type=jnp.float32)
    hmid = (g * jax.nn.sigmoid(g)) * u
    b_d = jax.lax.dot_general(hmid, da, (((1,), (1,)), ((), ())),
                              preferred_element_type=jnp.float32)
    dwn = jax.lax.dot_general(b_d, db, (((1,), (1,)), ((), ())),
                              preferred_element_type=jnp.float32)

    out_ref[...] += wcol * dwn


def kernel(hidden_states, Wg, gate_A, gate_B, up_A, up_B, down_A, down_B):
    x = hidden_states.reshape(-1, H)

    grid = (NT, E)
    out, logits = pl.pallas_call(
        _moe_body,
        grid=grid,
        in_specs=[
            pl.BlockSpec((TM, H), lambda i, e: (i, 0)),
            pl.BlockSpec((E, H), lambda i, e: (0, 0)),
            pl.BlockSpec((1, R, H), lambda i, e: (e, 0, 0)),
            pl.BlockSpec((1, I, R), lambda i, e: (e, 0, 0)),
            pl.BlockSpec((1, R, H), lambda i, e: (e, 0, 0)),
            pl.BlockSpec((1, I, R), lambda i, e: (e, 0, 0)),
            pl.BlockSpec((1, R, I), lambda i, e: (e, 0, 0)),
            pl.BlockSpec((1, H, R), lambda i, e: (e, 0, 0)),
        ],
        out_specs=[
            pl.BlockSpec((TM, H), lambda i, e: (i, 0)),
            pl.BlockSpec((TM, E), lambda i, e: (i, 0)),
        ],
        out_shape=[
            jax.ShapeDtypeStruct((S, H), jnp.float32),
            jax.ShapeDtypeStruct((S, E), jnp.float32),
        ],
        compiler_params=pltpu.CompilerParams(
            dimension_semantics=("parallel", "arbitrary"),
        ),
    )(x, Wg, gate_A, gate_B, up_A, up_B, down_A, down_B)

    return out.reshape(B, S, H), logits


# speedup vs baseline: 1.8078x; 1.1483x over previous
import jax
import jax.numpy as jnp
from jax.experimental import pallas as pl
from jax.experimental.pallas import tpu as pltpu

B, S, H = 1, 2048, 1024
I, R, E, K = 1024, 256, 8, 2
TM = 1024
NT = S // TM


def _moe_body(x_ref, wg_ref, ga_ref, gb_ref, ua_ref, ub_ref, da_ref, db_ref,
              out_ref, logits_ref):
    e = pl.program_id(1)
    x = x_ref[...]

    logits = jax.lax.dot_general(
        x, wg_ref[...], (((1,), (1,)), ((), ())),
        preferred_element_type=jnp.float32)
    p = jax.nn.softmax(logits, axis=-1)
    lane = jax.lax.broadcasted_iota(jnp.int32, (TM, E), 1)
    v1 = jnp.max(p, axis=-1, keepdims=True)
    idx1 = jnp.min(jnp.where(p == v1, lane, E), axis=-1, keepdims=True)
    p2 = jnp.where(lane == idx1, -1.0, p)
    v2 = jnp.max(p2, axis=-1, keepdims=True)
    idx2 = jnp.min(jnp.where(p2 == v2, lane, E), axis=-1, keepdims=True)
    denom = v1 + v2
    wcol = (jnp.where(idx1 == e, v1, 0.0) + jnp.where(idx2 == e, v2, 0.0)) / denom

    @pl.when(e == 0)
    def _():
        logits_ref[...] = logits
        out_ref[...] = jnp.zeros_like(out_ref)

    ga = ga_ref[0]
    gb = gb_ref[0]
    ua = ua_ref[0]
    ub = ub_ref[0]
    da = da_ref[0]
    db = db_ref[0]

    a_g = jax.lax.dot_general(x, ga, (((1,), (1,)), ((), ())),
                              preferred_element_type=jnp.float32)
    g = jax.lax.dot_general(a_g, gb, (((1,), (1,)), ((), ())),
                            preferred_element_type=jnp.float32)
    a_u = jax.lax.dot_general(x, ua, (((1,), (1,)), ((), ())),
                              preferred_element_type=jnp.float32)
    u = jax.lax.dot_general(a_u, ub, (((1,), (1,)), ((), ())),
                            preferred_element_type=jnp.float32)
    hmid = (g * jax.nn.sigmoid(g)) * u
    b_d = jax.lax.dot_general(hmid, da, (((1,), (1,)), ((), ())),
                              preferred_element_type=jnp.float32)
    dwn = jax.lax.dot_general(b_d, db, (((1,), (1,)), ((), ())),
                              preferred_element_type=jnp.float32)

    out_ref[...] += wcol * dwn


def kernel(hidden_states, Wg, gate_A, gate_B, up_A, up_B, down_A, down_B):
    x = hidden_states.reshape(-1, H)

    grid = (NT, E)
    out, logits = pl.pallas_call(
        _moe_body,
        grid=grid,
        in_specs=[
            pl.BlockSpec((TM, H), lambda i, e: (i, 0)),
            pl.BlockSpec((E, H), lambda i, e: (0, 0)),
            pl.BlockSpec((1, R, H), lambda i, e: (e, 0, 0)),
            pl.BlockSpec((1, I, R), lambda i, e: (e, 0, 0)),
            pl.BlockSpec((1, R, H), lambda i, e: (e, 0, 0)),
            pl.BlockSpec((1, I, R), lambda i, e: (e, 0, 0)),
            pl.BlockSpec((1, R, I), lambda i, e: (e, 0, 0)),
            pl.BlockSpec((1, H, R), lambda i, e: (e, 0, 0)),
        ],
        out_specs=[
            pl.BlockSpec((TM, H), lambda i, e: (i, 0)),
            pl.BlockSpec((TM, E), lambda i, e: (i, 0)),
        ],
        out_shape=[
            jax.ShapeDtypeStruct((S, H), jnp.float32),
            jax.ShapeDtypeStruct((S, E), jnp.float32),
        ],
        compiler_params=pltpu.CompilerParams(
            dimension_semantics=("parallel", "arbitrary"),
        ),
    )(x, Wg, gate_A, gate_B, up_A, up_B, down_A, down_B)

    return out.reshape(B, S, H), logits
